# baseline (speedup 1.0000x reference)
import functools

import jax
import jax.numpy as jnp
from jax import lax
from jax.experimental import pallas as pl
from jax.experimental.pallas import tpu as pltpu
from jax.experimental.pallas import tpu_sc as plsc

D = 32
W = 128
NC = 2
NS = 16
NW = NC * NS
CHUNK = 128
NBUF = 3
L = 16
BLK_R = 5000


def _merge_kernel(m_ref, r_ref, e_ref, o_ref):
    r = r_ref[...]
    std = jnp.maximum(r, 0.0) + jnp.log1p(jnp.exp(-jnp.abs(r)))
    zero = jnp.zeros_like(r)
    o_ref[...] = jnp.concatenate([m_ref[...], std, e_ref[...], zero], axis=1)


@jax.jit
def _tc_build(mean, rho, eps):
    vocab = mean.shape[0]
    spec_in = pl.BlockSpec((BLK_R, D), lambda i: (i, 0))
    return pl.pallas_call(
        _merge_kernel,
        grid=(vocab // BLK_R,),
        in_specs=[spec_in, spec_in, spec_in],
        out_specs=pl.BlockSpec((BLK_R, W), lambda i: (i, 0)),
        out_shape=jax.ShapeDtypeStruct((vocab, W), jnp.float32),
    )(mean, rho, eps)


@functools.partial(jax.jit, static_argnames=("total",))
def _sc_lookup(idx2d, merged, *, total):
    n_groups = idx2d.shape[0] // NW
    per_w = n_groups * CHUNK
    orows_w = per_w // 4
    orows_g = CHUNK // 4

    mesh = plsc.VectorSubcoreMesh(core_axis_name="c", subcore_axis_name="s")

    @functools.partial(
        pl.kernel,
        mesh=mesh,
        compiler_params=pltpu.CompilerParams(use_tc_tiling_on_sc=False),
        out_type=jax.ShapeDtypeStruct((total // 4, W), jnp.float32),
        scratch_types=[pltpu.VMEM((n_groups, CHUNK), jnp.int32)]
        + [pltpu.VMEM((CHUNK, W), jnp.float32) for _ in range(NBUF)]
        + [pltpu.VMEM((orows_g, W), jnp.float32) for _ in range(NBUF)]
        + [pltpu.SemaphoreType.DMA] * (2 * NBUF),
    )
    def body(idx_hbm, tab_hbm, out_hbm, idx_v, *scr):
        bufs = tuple(
            (scr[b], scr[NBUF + b],
             scr[2 * NBUF + 2 * b], scr[2 * NBUF + 2 * b + 1])
            for b in range(NBUF)
        )
        wid = lax.axis_index("s") * NC + lax.axis_index("c")
        pltpu.sync_copy(idx_hbm.at[pl.ds(wid * n_groups, n_groups)], idx_v)

        def gather_copy(g, b):
            g_v, _, gsem, _ = bufs[b]
            return pltpu.make_async_copy(tab_hbm.at[idx_v.at[g]], g_v, gsem)

        def out_copy(g, b):
            _, o_v, _, osem = bufs[b]
            return pltpu.make_async_copy(
                o_v, out_hbm.at[pl.ds(wid * orows_w + g * orows_g, orows_g)],
                osem)

        def compute(b):
            g_v, o_v, _, _ = bufs[b]

            @plsc.parallel_loop(0, CHUNK, unroll=4)
            def _(i):
                orow = i // 4
                ocol = (i % 4) * D
                for h in range(D // L):
                    m = g_v[i, pl.ds(h * L, L)]
                    s = g_v[i, pl.ds(D + h * L, L)]
                    e = g_v[i, pl.ds(2 * D + h * L, L)]
                    o_v[orow, pl.ds(ocol + h * L, L)] = m + s * e

        def step(g, b, *, first=False, prefetch=True):
            if prefetch:
                gather_copy(g + 2, (b + 2) % NBUF).start()
            gather_copy(g, b).wait()
            compute(b)
            out_copy(g, b).start()
            if not first:
                out_copy(g - 1, (b + 2) % NBUF).wait()

        gather_copy(0, 0).start()
        gather_copy(1, 1).start()
        step(0, 0, first=True)
        step(1, 1)

        def loop_body(t, carry):
            g = 2 + 3 * t
            step(g, 2)
            step(g + 1, 0)
            step(g + 2, 1)
            return carry

        lax.fori_loop(0, (n_groups - 5) // 3, loop_body, 0)

        step(n_groups - 3, (n_groups - 3) % NBUF)
        step(n_groups - 2, (n_groups - 2) % NBUF, prefetch=False)
        step(n_groups - 1, (n_groups - 1) % NBUF, prefetch=False)
        out_copy(n_groups - 1, (n_groups - 1) % NBUF).wait()

    return body(idx2d, merged)


def kernel(data, mean, rho, eps):
    batch, seq_len = data.shape
    total = batch * seq_len
    idx2d = data.reshape(total // CHUNK, CHUNK)
    merged = _tc_build(mean, rho, eps)
    out = _sc_lookup(idx2d, merged, total=total)
    return out.reshape(batch, seq_len, mean.shape[1])


# speedup vs baseline: 1.0058x; 1.0058x over previous
import functools

import jax
import jax.numpy as jnp
from jax import lax
from jax.experimental import pallas as pl
from jax.experimental.pallas import tpu as pltpu
from jax.experimental.pallas import tpu_sc as plsc

D = 32
W = 128
NC = 2
NS = 16
NW = NC * NS
CHUNK = 128
NBUF = 3
L = 16
BLK_R = 10000


def _merge_kernel(m_ref, r_ref, e_ref, o_ref):
    r = r_ref[...]
    std = jnp.maximum(r, 0.0) + jnp.log1p(jnp.exp(-jnp.abs(r)))
    zero = jnp.zeros_like(r)
    o_ref[...] = jnp.concatenate([m_ref[...], std, e_ref[...], zero], axis=1)


@jax.jit
def _tc_build(mean, rho, eps):
    vocab = mean.shape[0]
    spec_in = pl.BlockSpec((BLK_R, D), lambda i: (i, 0))
    return pl.pallas_call(
        _merge_kernel,
        grid=(vocab // BLK_R,),
        in_specs=[spec_in, spec_in, spec_in],
        out_specs=pl.BlockSpec((BLK_R, W), lambda i: (i, 0)),
        out_shape=jax.ShapeDtypeStruct((vocab, W), jnp.float32),
    )(mean, rho, eps)


@functools.partial(jax.jit, static_argnames=("total",))
def _sc_lookup(idx2d, merged, *, total):
    n_groups = idx2d.shape[0] // NW
    per_w = n_groups * CHUNK
    orows_w = per_w // 4
    orows_g = CHUNK // 4

    mesh = plsc.VectorSubcoreMesh(core_axis_name="c", subcore_axis_name="s")

    @functools.partial(
        pl.kernel,
        mesh=mesh,
        compiler_params=pltpu.CompilerParams(use_tc_tiling_on_sc=True),
        out_type=jax.ShapeDtypeStruct((total // 4, W), jnp.float32),
        scratch_types=[pltpu.VMEM((n_groups, CHUNK), jnp.int32)]
        + [pltpu.VMEM((CHUNK, W), jnp.float32) for _ in range(NBUF)]
        + [pltpu.VMEM((orows_g, W), jnp.float32) for _ in range(NBUF)]
        + [pltpu.SemaphoreType.DMA] * (2 * NBUF),
    )
    def body(idx_hbm, tab_hbm, out_hbm, idx_v, *scr):
        bufs = tuple(
            (scr[b], scr[NBUF + b],
             scr[2 * NBUF + 2 * b], scr[2 * NBUF + 2 * b + 1])
            for b in range(NBUF)
        )
        wid = lax.axis_index("s") * NC + lax.axis_index("c")
        pltpu.sync_copy(idx_hbm.at[pl.ds(wid * n_groups, n_groups)], idx_v)

        def gather_copy(g, b):
            g_v, _, gsem, _ = bufs[b]
            return pltpu.make_async_copy(tab_hbm.at[idx_v.at[g]], g_v, gsem)

        def out_copy(g, b):
            _, o_v, _, osem = bufs[b]
            return pltpu.make_async_copy(
                o_v, out_hbm.at[pl.ds(wid * orows_w + g * orows_g, orows_g)],
                osem)

        def compute(b):
            g_v, o_v, _, _ = bufs[b]

            @plsc.parallel_loop(0, CHUNK, unroll=4)
            def _(i):
                orow = i // 4
                ocol = (i % 4) * D
                for h in range(D // L):
                    m = g_v[i, pl.ds(h * L, L)]
                    s = g_v[i, pl.ds(D + h * L, L)]
                    e = g_v[i, pl.ds(2 * D + h * L, L)]
                    o_v[orow, pl.ds(ocol + h * L, L)] = m + s * e

        def step(g, b, *, first=False, prefetch=True):
            if prefetch:
                gather_copy(g + 2, (b + 2) % NBUF).start()
            gather_copy(g, b).wait()
            compute(b)
            out_copy(g, b).start()
            if not first:
                out_copy(g - 1, (b + 2) % NBUF).wait()

        gather_copy(0, 0).start()
        gather_copy(1, 1).start()
        step(0, 0, first=True)
        step(1, 1)

        def loop_body(t, carry):
            g = 2 + 3 * t
            step(g, 2)
            step(g + 1, 0)
            step(g + 2, 1)
            return carry

        lax.fori_loop(0, (n_groups - 5) // 3, loop_body, 0)

        step(n_groups - 3, (n_groups - 3) % NBUF)
        step(n_groups - 2, (n_groups - 2) % NBUF, prefetch=False)
        step(n_groups - 1, (n_groups - 1) % NBUF, prefetch=False)
        out_copy(n_groups - 1, (n_groups - 1) % NBUF).wait()

    return body(idx2d, merged)


def kernel(data, mean, rho, eps):
    batch, seq_len = data.shape
    total = batch * seq_len
    idx2d = data.reshape(total // CHUNK, CHUNK)
    merged = _tc_build(mean, rho, eps)
    out = _sc_lookup(idx2d, merged, total=total)
    return out.reshape(batch, seq_len, mean.shape[1])


# speedup vs baseline: 1.0986x; 1.0923x over previous
import functools

import jax
import jax.numpy as jnp
from jax import lax
from jax.experimental import pallas as pl
from jax.experimental.pallas import tpu as pltpu
from jax.experimental.pallas import tpu_sc as plsc

D = 32
W = 128
NC = 2
NS = 16
NW = NC * NS
CHUNK = 128
NBUF = 3
L = 16

_LOG1P = (-0.006151471, 0.03484971, -0.09325204, 0.16582276, -0.23982616,
          0.33154863, -0.49983856, 0.9999943, 3.3869654e-08)


@functools.partial(jax.jit, static_argnames=("total",))
def _sc_lookup(idx2d, mean, rho, eps, *, total):
    n_groups = idx2d.shape[0] // NW
    per_w = n_groups * CHUNK
    orows_w = per_w // 4
    orows_g = CHUNK // 4

    mesh = plsc.VectorSubcoreMesh(core_axis_name="c", subcore_axis_name="s")

    @functools.partial(
        pl.kernel,
        mesh=mesh,
        compiler_params=pltpu.CompilerParams(use_tc_tiling_on_sc=False),
        out_type=jax.ShapeDtypeStruct((total // 4, W), jnp.float32),
        scratch_types=[pltpu.VMEM((n_groups, CHUNK), jnp.int32)]
        + [pltpu.VMEM((CHUNK, D), jnp.float32) for _ in range(3 * NBUF)]
        + [pltpu.VMEM((orows_g, W), jnp.float32) for _ in range(NBUF)]
        + [pltpu.SemaphoreType.DMA] * (4 * NBUF),
    )
    def body(idx_hbm, m_hbm, r_hbm, e_hbm, out_hbm, idx_v, *scr):
        tabs = (m_hbm, r_hbm, e_hbm)
        bufs = tuple(
            (tuple(scr[3 * b + j] for j in range(3)),
             scr[3 * NBUF + b],
             tuple(scr[4 * NBUF + 4 * b + j] for j in range(3)),
             scr[4 * NBUF + 4 * b + 3])
            for b in range(NBUF)
        )
        wid = lax.axis_index("s") * NC + lax.axis_index("c")
        pltpu.sync_copy(idx_hbm.at[pl.ds(wid * n_groups, n_groups)], idx_v)

        def gather_copies(g, b):
            g_vs, _, gsems, _ = bufs[b]
            return tuple(
                pltpu.make_async_copy(tabs[j].at[idx_v.at[g]], g_vs[j], gsems[j])
                for j in range(3))

        def out_copy(g, b):
            _, o_v, _, osem = bufs[b]
            return pltpu.make_async_copy(
                o_v, out_hbm.at[pl.ds(wid * orows_w + g * orows_g, orows_g)],
                osem)

        def compute(b):
            (m_v, r_v, e_v), o_v, _, _ = bufs[b]

            @plsc.parallel_loop(0, CHUNK, unroll=4)
            def _(i):
                orow = i // 4
                ocol = (i % 4) * D
                for h in range(D // L):
                    m = m_v[i, pl.ds(h * L, L)]
                    r = r_v[i, pl.ds(h * L, L)]
                    e = e_v[i, pl.ds(h * L, L)]
                    t = jnp.exp(-jnp.abs(r))
                    p = jnp.full_like(t, _LOG1P[0])
                    for c in _LOG1P[1:]:
                        p = p * t + c
                    s = jnp.maximum(r, 0.0) + p
                    o_v[orow, pl.ds(ocol + h * L, L)] = m + s * e

        def step(g, b, *, first=False, prefetch=True):
            if prefetch:
                for c in gather_copies(g + 2, (b + 2) % NBUF):
                    c.start()
            for c in gather_copies(g, b):
                c.wait()
            compute(b)
            out_copy(g, b).start()
            if not first:
                out_copy(g - 1, (b + 2) % NBUF).wait()

        for c in gather_copies(0, 0):
            c.start()
        for c in gather_copies(1, 1):
            c.start()
        step(0, 0, first=True)
        step(1, 1)

        def loop_body(t, carry):
            g = 2 + 3 * t
            step(g, 2)
            step(g + 1, 0)
            step(g + 2, 1)
            return carry

        lax.fori_loop(0, (n_groups - 5) // 3, loop_body, 0)

        step(n_groups - 3, (n_groups - 3) % NBUF)
        step(n_groups - 2, (n_groups - 2) % NBUF, prefetch=False)
        step(n_groups - 1, (n_groups - 1) % NBUF, prefetch=False)
        out_copy(n_groups - 1, (n_groups - 1) % NBUF).wait()

    return body(idx2d, mean, rho, eps)


def kernel(data, mean, rho, eps):
    batch, seq_len = data.shape
    total = batch * seq_len
    idx2d = data.reshape(total // CHUNK, CHUNK)
    out = _sc_lookup(idx2d, mean, rho, eps, total=total)
    return out.reshape(batch, seq_len, mean.shape[1])
